# baseline (speedup 1.0000x reference)
import functools

import jax
import jax.numpy as jnp
from jax.experimental import pallas as pl
from jax.experimental.pallas import tpu as pltpu

_PAD_IDX = 0
_SUBLANE = 8
_CHUNK = 32
_SLOTS = 64
_SEG = 2048
_NBUF = 4
_MAX_TOKENS_PER_CALL = 32768


def _round_up(a, b):
    return (a + b - 1) // b * b


def _gather_kernel(idx_ref, w_hbm, out_hbm, buf, gsems, osems,
                   *, n_chunks, chunks_per_seg, nbuf, slots):
    ring_chunks = nbuf * chunks_per_seg
    n_segs = n_chunks // chunks_per_seg

    def issue(gi):
        dst_base = (gi % ring_chunks) * _CHUNK
        src_base = gi * _CHUNK
        slot = gi % slots
        for k in range(_CHUNK):
            row = idx_ref[src_base + k]
            pltpu.make_async_copy(
                w_hbm.at[pl.ds(row, 1), :],
                buf.at[pl.ds(dst_base + k, 1), :],
                gsems.at[slot],
            ).start(priority=k % 2)

    def wait_chunk(g):
        pltpu.make_async_copy(
            buf.at[pl.ds(0, _CHUNK), :],
            buf.at[pl.ds(0, _CHUNK), :],
            gsems.at[g % slots],
        ).wait()

    def start_out(seg):
        pltpu.make_async_copy(
            buf.at[pl.ds((seg % nbuf) * _SEG, _SEG), :],
            out_hbm.at[pl.ds(seg * _SEG, _SEG), :],
            osems.at[seg % nbuf],
        ).start()

    def wait_out(b):
        pltpu.make_async_copy(
            buf.at[pl.ds(0, _SEG), :],
            out_hbm.at[pl.ds(0, _SEG), :],
            osems.at[b],
        ).wait()

    depth = min(slots - 1, n_chunks)

    def pro_body(j, _):
        issue(j)
        return _
    jax.lax.fori_loop(0, depth, pro_body, 0)

    def body(g, _):
        gi = g + depth

        @pl.when(gi < n_chunks)
        def _issue():
            seg_i = gi // chunks_per_seg

            @pl.when(jnp.logical_and(gi % chunks_per_seg == 0,
                                     seg_i >= nbuf))
            def _reuse():
                wait_out((seg_i - nbuf) % nbuf)

            issue(gi)

        wait_chunk(g)

        @pl.when(g % chunks_per_seg == chunks_per_seg - 1)
        def _flush():
            start_out(g // chunks_per_seg)

        return _
    jax.lax.fori_loop(0, n_chunks, body, 0)

    for b in range(min(nbuf, n_segs)):
        wait_out((max(n_segs - nbuf, 0) + b) % nbuf)


def _lookup_hbm_gather(flat_ids, weights):
    n_tok = flat_ids.shape[0]
    if n_tok > _MAX_TOKENS_PER_CALL:
        parts = [
            _lookup_hbm_gather(flat_ids[s:s + _MAX_TOKENS_PER_CALL], weights)
            for s in range(0, n_tok, _MAX_TOKENS_PER_CALL)
        ]
        return jnp.concatenate(parts, axis=0)

    vsz, dsz = weights.shape
    n_pad = _round_up(n_tok, _SEG)
    n_chunks = n_pad // _CHUNK
    chunks_per_seg = _SEG // _CHUNK
    n_segs = n_pad // _SEG
    nbuf = min(_NBUF, n_segs)

    ids = jnp.pad(flat_ids, (0, n_pad - n_tok), constant_values=_PAD_IDX)

    out = pl.pallas_call(
        functools.partial(_gather_kernel, n_chunks=n_chunks,
                          chunks_per_seg=chunks_per_seg, nbuf=nbuf,
                          slots=_SLOTS),
        out_shape=jax.ShapeDtypeStruct((n_pad, dsz), weights.dtype),
        grid_spec=pltpu.PrefetchScalarGridSpec(
            num_scalar_prefetch=1,
            grid=(1,),
            in_specs=[pl.BlockSpec(memory_space=pl.ANY)],
            out_specs=pl.BlockSpec(memory_space=pl.ANY),
            scratch_shapes=[
                pltpu.VMEM((_NBUF * _SEG, dsz), weights.dtype),
                pltpu.SemaphoreType.DMA((_SLOTS,)),
                pltpu.SemaphoreType.DMA((_NBUF,)),
            ],
        ),
        compiler_params=pltpu.CompilerParams(
            dimension_semantics=("arbitrary",),
        ),
    )(ids, weights)
    return out[:n_tok]


def kernel(x, weights):
    B, T = x.shape
    vsz, dsz = weights.shape

    flat_ids = jnp.clip(x.reshape(-1).astype(jnp.int32), 0, vsz - 1)

    out_flat = _lookup_hbm_gather(flat_ids, weights)
    return out_flat.reshape(B, T, dsz)


# speedup vs baseline: 1.6559x; 1.6559x over previous
import functools

import jax
import jax.numpy as jnp
from jax.experimental import pallas as pl
from jax.experimental.pallas import tpu as pltpu

_PAD_IDX = 0
_SUBLANE = 8
_TB = 4096
_CHUNK = 32
_SLOTS = 64
_MAX_TOKENS_PER_CALL = 32768


def _round_up(a, b):
    return (a + b - 1) // b * b


def _gather_kernel(idx_ref, w_hbm, out_ref, sems, *, tb, chunk, slots):
    base = pl.program_id(0) * tb
    n_chunks = tb // chunk

    def issue(c):
        slot = c % slots
        for k in range(chunk):
            r = c * chunk + k
            row = idx_ref[base + r]
            pltpu.make_async_copy(
                w_hbm.at[pl.ds(row, 1), :],
                out_ref.at[pl.ds(r, 1), :],
                sems.at[slot],
            ).start(priority=c % 2)

    def wait(c):
        pltpu.make_async_copy(
            w_hbm.at[pl.ds(0, chunk), :],
            out_ref.at[pl.ds(c * chunk, chunk), :],
            sems.at[c % slots],
        ).wait()

    depth = min(slots - 1, n_chunks)
    for c in range(depth):
        issue(c)
    for c in range(n_chunks):
        if c + depth < n_chunks:
            issue(c + depth)
        wait(c)


def _lookup_hbm_gather(flat_ids, weights, tb):
    n_tok = flat_ids.shape[0]
    if n_tok > _MAX_TOKENS_PER_CALL:
        parts = [
            _lookup_hbm_gather(flat_ids[s:s + _MAX_TOKENS_PER_CALL], weights, tb)
            for s in range(0, n_tok, _MAX_TOKENS_PER_CALL)
        ]
        return jnp.concatenate(parts, axis=0)

    vsz, dsz = weights.shape
    n_pad = _round_up(n_tok, tb)
    nb = n_pad // tb
    if tb % _CHUNK == 0:
        chunk = _CHUNK
    elif tb % 32 == 0:
        chunk = 32
    else:
        chunk = _SUBLANE

    ids = jnp.pad(flat_ids, (0, n_pad - n_tok), constant_values=_PAD_IDX)

    out = pl.pallas_call(
        functools.partial(_gather_kernel, tb=tb, chunk=chunk, slots=_SLOTS),
        out_shape=jax.ShapeDtypeStruct((n_pad, dsz), weights.dtype),
        grid_spec=pltpu.PrefetchScalarGridSpec(
            num_scalar_prefetch=1,
            grid=(nb,),
            in_specs=[pl.BlockSpec(memory_space=pl.ANY)],
            out_specs=pl.BlockSpec((tb, dsz), lambda i, idx: (i, 0)),
            scratch_shapes=[pltpu.SemaphoreType.DMA((_SLOTS,))],
        ),
        compiler_params=pltpu.CompilerParams(
            dimension_semantics=("parallel",),
        ),
    )(ids, weights)
    return out[:n_tok]


def kernel(x, weights):
    B, T = x.shape
    vsz, dsz = weights.shape

    flat_ids = jnp.clip(x.reshape(-1).astype(jnp.int32), 0, vsz - 1)
    n_tok = flat_ids.shape[0]

    tb = _round_up(min(_TB, _round_up(n_tok, _SUBLANE)), _SUBLANE)
    out_flat = _lookup_hbm_gather(flat_ids, weights, tb)
    return out_flat.reshape(B, T, dsz)
